# baseline (speedup 1.0000x reference)
import numpy as np
import jax
import jax.numpy as jnp
from jax.experimental import pallas as pl
from jax.experimental.pallas import tpu as pltpu

_DIM = 16
_K = 8
_TB = 3.0
_MIN_BW = 1e-3
_MIN_BH = 1e-3
_MIN_D = 1e-3
_PAD_C = float(np.log(np.exp(1 - _MIN_D) - 1))

_BL = 16384
_CHUNK = 128


def _edges_from(u, min_b):
    m = jnp.max(u, axis=1, keepdims=True)
    e = jnp.exp(u - m)
    w = min_b + (1 - min_b * _K) * (e / jnp.sum(e, axis=1, keepdims=True))
    edges = [jnp.full((_DIM, 1), -_TB, dtype=u.dtype)]
    acc = jnp.zeros((_DIM, 1), dtype=u.dtype)
    for k in range(_K - 1):
        acc = acc + w[:, k : k + 1]
        edges.append(2 * _TB * acc - _TB)
    edges.append(jnp.full((_DIM, 1), _TB, dtype=u.dtype))
    return edges


def _body(uw_ref, uh_ref, ud_ref, x_ref, out_ref, lad_ref):
    f32 = jnp.float32
    ew = _edges_from(uw_ref[...], _MIN_BW)
    eh = _edges_from(uh_ref[...], _MIN_BH)
    widths = [ew[k + 1] - ew[k] for k in range(_K)]
    heights = [eh[k + 1] - eh[k] for k in range(_K)]
    rw = [1.0 / widths[k] for k in range(_K)]

    ud = ud_ref[...]
    pad = jnp.full((_DIM, 1), _PAD_C, dtype=f32)
    ud_cols = [pad] + [ud[:, k : k + 1] for k in range(_K - 1)] + [pad]
    derivs = [_MIN_D + jnp.log1p(jnp.exp(u)) for u in ud_cols]

    def chunk(j, carry):
        base = pl.multiple_of(j * _CHUNK, _CHUNK)
        x = x_ref[:, pl.ds(base, _CHUNK)]
        inside = (x >= -_TB) & (x <= _TB)
        x_in = jnp.clip(x, -_TB, _TB)

        g_cumw = jnp.broadcast_to(ew[0], x.shape)
        g_rw = jnp.broadcast_to(rw[0], x.shape)
        g_h = jnp.broadcast_to(heights[0], x.shape)
        g_cumh = jnp.broadcast_to(eh[0], x.shape)
        g_d = jnp.broadcast_to(derivs[0], x.shape)
        g_d1 = jnp.broadcast_to(derivs[1], x.shape)
        for k in range(1, _K):
            m = (x_in >= ew[k]).astype(f32)
            g_cumw = g_cumw + (ew[k] - ew[k - 1]) * m
            g_rw = g_rw + (rw[k] - rw[k - 1]) * m
            g_h = g_h + (heights[k] - heights[k - 1]) * m
            g_cumh = g_cumh + (eh[k] - eh[k - 1]) * m
            g_d = g_d + (derivs[k] - derivs[k - 1]) * m
            g_d1 = g_d1 + (derivs[k + 1] - derivs[k]) * m

        g_delta = g_h * g_rw
        theta = (x_in - g_cumw) * g_rw
        omt = 1.0 - theta
        tomt = theta * omt
        th2 = theta * theta
        num = g_h * (g_delta * th2 + g_d * tomt)
        den = g_delta + (g_d + g_d1 - 2.0 * g_delta) * tomt
        rden = 1.0 / den
        out_in = g_cumh + num * rden
        dnum = (g_delta * g_delta) * (
            g_d1 * th2 + 2.0 * g_delta * tomt + g_d * (omt * omt))
        lad_in = jnp.log(dnum * rden * rden)

        out_ref[:, pl.ds(base, _CHUNK)] = jnp.where(inside, out_in, x)
        lad_ref[:, pl.ds(base, _CHUNK)] = jnp.where(inside, lad_in, 0.0)
        return carry

    jax.lax.fori_loop(0, _BL // _CHUNK, chunk, 0)


def kernel(x, unnormalized_widths, unnormalized_heights, unnormalized_derivatives):
    n, d = x.shape
    xt = x.T

    grid = (n // _BL,)
    out_t, lad_t = pl.pallas_call(
        _body,
        grid=grid,
        in_specs=[
            pl.BlockSpec((_DIM, _K), lambda i: (0, 0)),
            pl.BlockSpec((_DIM, _K), lambda i: (0, 0)),
            pl.BlockSpec((_DIM, _K - 1), lambda i: (0, 0)),
            pl.BlockSpec((_DIM, _BL), lambda i: (0, i)),
        ],
        out_specs=[
            pl.BlockSpec((_DIM, _BL), lambda i: (0, i)),
            pl.BlockSpec((_DIM, _BL), lambda i: (0, i)),
        ],
        out_shape=[
            jax.ShapeDtypeStruct((d, n), jnp.float32),
            jax.ShapeDtypeStruct((d, n), jnp.float32),
        ],
        compiler_params=pltpu.CompilerParams(
            dimension_semantics=("arbitrary",),
        ),
    )(unnormalized_widths, unnormalized_heights, unnormalized_derivatives, xt)
    return out_t.T, lad_t.T


# speedup vs baseline: 2.7922x; 2.7922x over previous
import numpy as np
import jax
import jax.numpy as jnp
from jax.experimental import pallas as pl
from jax.experimental.pallas import tpu as pltpu

_DIM = 16
_K = 8
_TB = 3.0
_MIN_BW = 1e-3
_MIN_BH = 1e-3
_MIN_D = 1e-3
_PAD_C = float(np.log(np.exp(1 - _MIN_D) - 1))

_BL = 8192
_CHUNK = 128


def _edges_from(u, min_b):
    m = jnp.max(u, axis=1, keepdims=True)
    e = jnp.exp(u - m)
    w = min_b + (1 - min_b * _K) * (e / jnp.sum(e, axis=1, keepdims=True))
    edges = [jnp.full((_DIM, 1), -_TB, dtype=u.dtype)]
    acc = jnp.zeros((_DIM, 1), dtype=u.dtype)
    for k in range(_K - 1):
        acc = acc + w[:, k : k + 1]
        edges.append(2 * _TB * acc - _TB)
    edges.append(jnp.full((_DIM, 1), _TB, dtype=u.dtype))
    return edges


def _body(uw_ref, uh_ref, ud_ref, x_ref, out_ref, lad_ref):
    f32 = jnp.float32
    ew = _edges_from(uw_ref[...], _MIN_BW)
    eh = _edges_from(uh_ref[...], _MIN_BH)
    widths = [ew[k + 1] - ew[k] for k in range(_K)]
    heights = [eh[k + 1] - eh[k] for k in range(_K)]
    rw = [1.0 / widths[k] for k in range(_K)]

    ud = ud_ref[...]
    pad = jnp.full((_DIM, 1), _PAD_C, dtype=f32)
    ud_cols = [pad] + [ud[:, k : k + 1] for k in range(_K - 1)] + [pad]
    derivs = [_MIN_D + jnp.log1p(jnp.exp(u)) for u in ud_cols]

    x = x_ref[...]
    inside = (x >= -_TB) & (x <= _TB)
    x_in = jnp.clip(x, -_TB, _TB)

    g_cumw = jnp.broadcast_to(ew[0], x.shape)
    g_rw = jnp.broadcast_to(rw[0], x.shape)
    g_h = jnp.broadcast_to(heights[0], x.shape)
    g_cumh = jnp.broadcast_to(eh[0], x.shape)
    g_d = jnp.broadcast_to(derivs[0], x.shape)
    g_d1 = jnp.broadcast_to(derivs[1], x.shape)
    for j in range(1, _K):
        m = (x_in >= ew[j]).astype(f32)
        g_cumw = g_cumw + (ew[j] - ew[j - 1]) * m
        g_rw = g_rw + (rw[j] - rw[j - 1]) * m
        g_h = g_h + (heights[j] - heights[j - 1]) * m
        g_cumh = g_cumh + (eh[j] - eh[j - 1]) * m
        g_d = g_d + (derivs[j] - derivs[j - 1]) * m
        g_d1 = g_d1 + (derivs[j + 1] - derivs[j]) * m

    g_delta = g_h * g_rw
    theta = (x_in - g_cumw) * g_rw
    omt = 1.0 - theta
    tomt = theta * omt
    th2 = theta * theta
    num = g_h * (g_delta * th2 + g_d * tomt)
    den = g_delta + (g_d + g_d1 - 2.0 * g_delta) * tomt
    rden = 1.0 / den
    out_in = g_cumh + num * rden
    dnum = (g_delta * g_delta) * (g_d1 * th2 + 2.0 * g_delta * tomt + g_d * (omt * omt))
    lad_in = jnp.log(dnum * rden * rden)

    out_ref[...] = jnp.where(inside, out_in, x)
    lad_ref[...] = jnp.where(inside, lad_in, 0.0)


def kernel(x, unnormalized_widths, unnormalized_heights, unnormalized_derivatives):
    n, d = x.shape
    xt = x.T

    grid = (n // _BL,)
    out_t, lad_t = pl.pallas_call(
        _body,
        grid=grid,
        in_specs=[
            pl.BlockSpec((_DIM, _K), lambda i: (0, 0)),
            pl.BlockSpec((_DIM, _K), lambda i: (0, 0)),
            pl.BlockSpec((_DIM, _K - 1), lambda i: (0, 0)),
            pl.BlockSpec((_DIM, _BL), lambda i: (0, i)),
        ],
        out_specs=[
            pl.BlockSpec((_DIM, _BL), lambda i: (0, i)),
            pl.BlockSpec((_DIM, _BL), lambda i: (0, i)),
        ],
        out_shape=[
            jax.ShapeDtypeStruct((d, n), jnp.float32),
            jax.ShapeDtypeStruct((d, n), jnp.float32),
        ],
        compiler_params=pltpu.CompilerParams(
            dimension_semantics=("arbitrary",),
        ),
    )(unnormalized_widths, unnormalized_heights, unnormalized_derivatives, xt)
    return out_t.T, lad_t.T


# speedup vs baseline: 3.5796x; 1.2820x over previous
import numpy as np
import jax
import jax.numpy as jnp
from jax.experimental import pallas as pl
from jax.experimental.pallas import tpu as pltpu

_DIM = 16
_K = 8
_TB = 3.0
_MIN_BW = 1e-3
_MIN_BH = 1e-3
_MIN_D = 1e-3
_PAD_C = float(np.log(np.exp(1 - _MIN_D) - 1))

_BL = 4096
_CHUNK = 128


def _edges_from(u, min_b):
    m = jnp.max(u, axis=1, keepdims=True)
    e = jnp.exp(u - m)
    w = min_b + (1 - min_b * _K) * (e / jnp.sum(e, axis=1, keepdims=True))
    edges = [jnp.full((_DIM, 1), -_TB, dtype=u.dtype)]
    acc = jnp.zeros((_DIM, 1), dtype=u.dtype)
    for k in range(_K - 1):
        acc = acc + w[:, k : k + 1]
        edges.append(2 * _TB * acc - _TB)
    edges.append(jnp.full((_DIM, 1), _TB, dtype=u.dtype))
    return edges


def _body(uw_ref, uh_ref, ud_ref, x_ref, out_ref, lad_ref):
    f32 = jnp.float32
    ew = _edges_from(uw_ref[...], _MIN_BW)
    eh = _edges_from(uh_ref[...], _MIN_BH)
    widths = [ew[k + 1] - ew[k] for k in range(_K)]
    heights = [eh[k + 1] - eh[k] for k in range(_K)]
    rw = [1.0 / widths[k] for k in range(_K)]

    ud = ud_ref[...]
    pad = jnp.full((_DIM, 1), _PAD_C, dtype=f32)
    ud_cols = [pad] + [ud[:, k : k + 1] for k in range(_K - 1)] + [pad]
    derivs = [_MIN_D + jnp.log1p(jnp.exp(u)) for u in ud_cols]

    x = x_ref[...]
    inside = (x >= -_TB) & (x <= _TB)
    x_in = jnp.clip(x, -_TB, _TB)

    g_cumw = jnp.broadcast_to(ew[0], x.shape)
    g_rw = jnp.broadcast_to(rw[0], x.shape)
    g_h = jnp.broadcast_to(heights[0], x.shape)
    g_cumh = jnp.broadcast_to(eh[0], x.shape)
    g_d = jnp.broadcast_to(derivs[0], x.shape)
    g_d1 = jnp.broadcast_to(derivs[1], x.shape)
    for j in range(1, _K):
        m = (x_in >= ew[j]).astype(f32)
        g_cumw = g_cumw + (ew[j] - ew[j - 1]) * m
        g_rw = g_rw + (rw[j] - rw[j - 1]) * m
        g_h = g_h + (heights[j] - heights[j - 1]) * m
        g_cumh = g_cumh + (eh[j] - eh[j - 1]) * m
        g_d = g_d + (derivs[j] - derivs[j - 1]) * m
        g_d1 = g_d1 + (derivs[j + 1] - derivs[j]) * m

    g_delta = g_h * g_rw
    theta = (x_in - g_cumw) * g_rw
    omt = 1.0 - theta
    tomt = theta * omt
    th2 = theta * theta
    num = g_h * (g_delta * th2 + g_d * tomt)
    den = g_delta + (g_d + g_d1 - 2.0 * g_delta) * tomt
    rden = 1.0 / den
    out_in = g_cumh + num * rden
    dnum = (g_delta * g_delta) * (g_d1 * th2 + 2.0 * g_delta * tomt + g_d * (omt * omt))
    lad_in = jnp.log(dnum * rden * rden)

    out_ref[...] = jnp.where(inside, out_in, x)
    lad_ref[...] = jnp.where(inside, lad_in, 0.0)


def kernel(x, unnormalized_widths, unnormalized_heights, unnormalized_derivatives):
    n, d = x.shape
    xt = x.T

    grid = (n // _BL,)
    out_t, lad_t = pl.pallas_call(
        _body,
        grid=grid,
        in_specs=[
            pl.BlockSpec((_DIM, _K), lambda i: (0, 0)),
            pl.BlockSpec((_DIM, _K), lambda i: (0, 0)),
            pl.BlockSpec((_DIM, _K - 1), lambda i: (0, 0)),
            pl.BlockSpec((_DIM, _BL), lambda i: (0, i)),
        ],
        out_specs=[
            pl.BlockSpec((_DIM, _BL), lambda i: (0, i)),
            pl.BlockSpec((_DIM, _BL), lambda i: (0, i)),
        ],
        out_shape=[
            jax.ShapeDtypeStruct((d, n), jnp.float32),
            jax.ShapeDtypeStruct((d, n), jnp.float32),
        ],
        compiler_params=pltpu.CompilerParams(
            dimension_semantics=("arbitrary",),
        ),
    )(unnormalized_widths, unnormalized_heights, unnormalized_derivatives, xt)
    return out_t.T, lad_t.T
